# baseline (speedup 1.0000x reference)
import functools

import jax
import jax.numpy as jnp
from jax import lax
from jax.experimental import pallas as pl
from jax.experimental.pallas import tpu as pltpu
from jax.experimental.pallas import tpu_sc as plsc


def _pack_bf16(a):
    u = lax.bitcast_convert_type(a.astype(jnp.bfloat16), jnp.uint16)
    m = u.shape[1] // 2
    lo = u[:, :m].astype(jnp.int32)
    hi = u[:, m:].astype(jnp.int32)
    return lo | (hi << 16)


def _unpack_bf16(p):
    lo = (p & 0xFFFF).astype(jnp.uint16)
    hi = lax.shift_right_logical(p, 16).astype(jnp.uint16)
    return jnp.concatenate([lax.bitcast_convert_type(lo, jnp.bfloat16),
                            lax.bitcast_convert_type(hi, jnp.bfloat16)],
                           axis=1)


D_MODEL = 1024
D_EXPERT = 512
NUM_EXPERTS = 8
TOPK = 2
T = 8192
NH = 2
TH = T // NH
PH = TH * TOPK
BT = 512
BC = 512
NBLK = PH // BC + NUM_EXPERTS
BUF = NBLK * BC
NW = 32
PW = PH // NW
CH = 64
HB = TH // BT


def _router_block(x_ref, wg_ref, e0_ref, e1_ref, r0_ref, r1_ref,
                  w0_ref, w1_ref, cnt_ref, xb16_ref, crun_ref):
    i = pl.program_id(0)

    @pl.when(lax.rem(i, HB) == 0)
    def _():
        crun_ref[...] = jnp.zeros((8, NUM_EXPERTS), jnp.float32)

    xb = x_ref[...]
    logits = lax.dot_general(xb, wg_ref[...], (((1,), (1,)), ((), ())),
                             preferred_element_type=jnp.float32)
    m = jnp.max(logits, axis=-1, keepdims=True)
    ex = jnp.exp(logits - m)
    probs = ex / jnp.sum(ex, axis=-1, keepdims=True)

    e0 = jnp.argmax(probs, axis=-1)
    w0 = jnp.max(probs, axis=-1)
    iota = lax.broadcasted_iota(jnp.int32, probs.shape, 1)
    probs2 = jnp.where(iota == e0[:, None], -jnp.inf, probs)
    e1 = jnp.argmax(probs2, axis=-1)
    w1 = jnp.max(probs2, axis=-1)
    s = w0 + w1

    oh0 = (iota == e0[:, None]).astype(jnp.float32)
    oh1 = (iota == e1[:, None]).astype(jnp.float32)
    ri = lax.broadcasted_iota(jnp.int32, (BT, BT), 0)
    ci = lax.broadcasted_iota(jnp.int32, (BT, BT), 1)
    tri = (ri > ci).astype(jnp.float32)
    ex0 = lax.dot_general(tri, oh0, (((1,), (0,)), ((), ())),
                          preferred_element_type=jnp.float32)
    ex1 = lax.dot_general(tri, oh1, (((1,), (0,)), ((), ())),
                          preferred_element_type=jnp.float32)
    cnt0 = jnp.sum(oh0, axis=0)
    cnt1 = jnp.sum(oh1, axis=0)
    crun = crun_ref[0:1, :]
    r0 = jnp.sum(oh0 * (crun + ex0), axis=1)
    r1 = jnp.sum(oh1 * (crun + cnt0[None, :] + ex1), axis=1)
    new = crun[0] + cnt0 + cnt1
    crun_ref[...] = jnp.broadcast_to(new[None, :], (8, NUM_EXPERTS))
    cnt_ref[...] = jnp.broadcast_to(new[None, :], (8, NUM_EXPERTS))

    e0_ref[...] = e0[:, None]
    e1_ref[...] = e1[:, None]
    r0_ref[...] = r0.astype(jnp.int32)[:, None]
    r1_ref[...] = r1.astype(jnp.int32)[:, None]
    w0_ref[...] = (w0 / s)[:, None]
    w1_ref[...] = (w1 / s)[:, None]
    xb16_ref[...] = _pack_bf16(xb)


def _router(xf, Wg):
    shapes = [
        jax.ShapeDtypeStruct((T, 1), jnp.int32),
        jax.ShapeDtypeStruct((T, 1), jnp.int32),
        jax.ShapeDtypeStruct((T, 1), jnp.int32),
        jax.ShapeDtypeStruct((T, 1), jnp.int32),
        jax.ShapeDtypeStruct((T, 1), jnp.float32),
        jax.ShapeDtypeStruct((T, 1), jnp.float32),
        jax.ShapeDtypeStruct((8 * NH, NUM_EXPERTS), jnp.float32),
        jax.ShapeDtypeStruct((T, D_MODEL // 2), jnp.int32),
    ]
    tspec = pl.BlockSpec((BT, 1), lambda i: (i, 0))
    return pl.pallas_call(
        _router_block,
        grid=(T // BT,),
        in_specs=[
            pl.BlockSpec((BT, D_MODEL), lambda i: (i, 0)),
            pl.BlockSpec((NUM_EXPERTS, D_MODEL), lambda i: (0, 0)),
        ],
        out_specs=[tspec, tspec, tspec, tspec, tspec, tspec,
                   pl.BlockSpec((8, NUM_EXPERTS), lambda i: (i // HB, 0)),
                   pl.BlockSpec((BT, D_MODEL // 2), lambda i: (i, 0))],
        out_shape=shapes,
        scratch_shapes=[pltpu.VMEM((8, NUM_EXPERTS), jnp.float32)],
    )(xf, Wg)


def _sc_scatter(xp, p, h):
    mesh = plsc.VectorSubcoreMesh(core_axis_name="c", subcore_axis_name="s")

    @functools.partial(
        pl.kernel, mesh=mesh,
        out_type=jax.ShapeDtypeStruct((BUF, D_MODEL // 2), jnp.int32),
        scratch_types=[pltpu.VMEM((CH,), jnp.int32),
                       pltpu.VMEM((CH, D_MODEL // 2), jnp.int32)],
    )
    def k(x_hbm, p_hbm, buf_hbm, idx_v, data_v):
        wid = lax.axis_index("s") * 2 + lax.axis_index("c")
        base = wid * PW
        trow = h * TH + lax.rem(base, TH)

        @pl.loop(0, PW // CH)
        def _(c):
            pltpu.sync_copy(x_hbm.at[pl.ds(trow + c * CH, CH)], data_v)
            pltpu.sync_copy(p_hbm.at[pl.ds(base + c * CH, CH)], idx_v)
            pltpu.sync_copy(data_v, buf_hbm.at[idx_v])

    return k(xp, p)


def _sc_gather(y, p):
    mesh = plsc.VectorSubcoreMesh(core_axis_name="c", subcore_axis_name="s")

    @functools.partial(
        pl.kernel, mesh=mesh,
        out_type=jax.ShapeDtypeStruct((PH, D_MODEL // 2), jnp.int32),
        scratch_types=[pltpu.VMEM((CH,), jnp.int32),
                       pltpu.VMEM((CH, D_MODEL // 2), jnp.int32),
                       pltpu.SemaphoreType.DMA],
    )
    def k(y_hbm, p_hbm, g_hbm, idx_v, rows_v, sem):
        wid = lax.axis_index("s") * 2 + lax.axis_index("c")
        base = wid * PW

        @pl.loop(0, PW // CH)
        def _(c):
            pltpu.sync_copy(p_hbm.at[pl.ds(base + c * CH, CH)], idx_v)
            pltpu.async_copy(y_hbm.at[idx_v], rows_v, sem).wait()
            pltpu.sync_copy(rows_v, g_hbm.at[pl.ds(base + c * CH, CH)])

    return k(y, p)


def _ffn_block(be_ref, buf_ref, w1_ref, b1_ref, w2_ref, b2_ref, y_ref):
    xb = _unpack_bf16(buf_ref[...])
    h = lax.dot_general(xb, w1_ref[0].astype(jnp.bfloat16),
                        (((1,), (1,)), ((), ())),
                        preferred_element_type=jnp.float32) + b1_ref[0]
    h = jnp.maximum(h, 0.0).astype(jnp.bfloat16)
    y = lax.dot_general(h, w2_ref[0].astype(jnp.bfloat16),
                        (((1,), (1,)), ((), ())),
                        preferred_element_type=jnp.float32) + b2_ref[0]
    y_ref[...] = _pack_bf16(y)


def _grouped_ffn(be, buf, W1, b1, W2, b2):
    grid_spec = pltpu.PrefetchScalarGridSpec(
        num_scalar_prefetch=1,
        grid=(NBLK,),
        in_specs=[
            pl.BlockSpec((BC, D_MODEL // 2), lambda i, be: (i, 0)),
            pl.BlockSpec((1, D_EXPERT, D_MODEL), lambda i, be: (be[i], 0, 0)),
            pl.BlockSpec((1, 1, D_EXPERT), lambda i, be: (be[i], 0, 0)),
            pl.BlockSpec((1, D_MODEL, D_EXPERT), lambda i, be: (be[i], 0, 0)),
            pl.BlockSpec((1, 1, D_MODEL), lambda i, be: (be[i], 0, 0)),
        ],
        out_specs=pl.BlockSpec((BC, D_MODEL // 2), lambda i, be: (i, 0)),
    )
    return pl.pallas_call(
        _ffn_block,
        grid_spec=grid_spec,
        out_shape=jax.ShapeDtypeStruct((BUF, D_MODEL // 2), jnp.int32),
    )(be, buf, W1, b1.reshape(NUM_EXPERTS, 1, D_EXPERT),
      W2, b2.reshape(NUM_EXPERTS, 1, D_MODEL))


def _combine_block(ya_ref, yb_ref, w0_ref, w1_ref, out_ref):
    ya = _unpack_bf16(ya_ref[...]).astype(jnp.float32)
    yb = _unpack_bf16(yb_ref[...]).astype(jnp.float32)
    out_ref[...] = ya * w0_ref[...] + yb * w1_ref[...]


def _combine(g, w0, w1, h):
    nb = TH // BT
    return pl.pallas_call(
        _combine_block,
        grid=(nb,),
        in_specs=[
            pl.BlockSpec((BT, D_MODEL // 2), lambda i: (i, 0)),
            pl.BlockSpec((BT, D_MODEL // 2), lambda i: (i + nb, 0)),
            pl.BlockSpec((BT, 1), lambda i: (i + h * nb, 0)),
            pl.BlockSpec((BT, 1), lambda i: (i + h * nb, 0)),
        ],
        out_specs=pl.BlockSpec((BT, D_MODEL), lambda i: (i, 0)),
        out_shape=jax.ShapeDtypeStruct((TH, D_MODEL), jnp.float32),
    )(g, g, w0, w1)


def _half_plumbing(counts, e0, e1, r0, r1, h):
    cnt = counts[8 * h].astype(jnp.int32)
    padded = ((cnt + BC - 1) // BC) * BC
    ends = jnp.cumsum(padded)
    off = ends - padded
    sl = slice(h * TH, (h + 1) * TH)
    p0 = jnp.take(off, e0[sl, 0]) + r0[sl, 0]
    p1 = jnp.take(off, e1[sl, 0]) + r1[sl, 0]
    p = jnp.concatenate([p0, p1])
    starts = jnp.arange(NBLK, dtype=jnp.int32) * BC
    be = jnp.minimum(
        jnp.searchsorted(ends, starts, side="right").astype(jnp.int32),
        NUM_EXPERTS - 1)
    return p, be


@jax.jit
def kernel(x, Wg, W1, b1, W2, b2):
    B, S, D = x.shape
    xf = x.reshape(T, D)

    e0, e1, r0, r1, w0, w1, counts, xp = _router(xf, Wg)
    p0h, be0 = _half_plumbing(counts, e0, e1, r0, r1, 0)
    p1h, be1 = _half_plumbing(counts, e0, e1, r0, r1, 1)

    buf0 = _sc_scatter(xp, p0h, 0)
    buf1 = _sc_scatter(xp, p1h, 1)
    y0 = _grouped_ffn(be0, buf0, W1, b1, W2, b2)
    y1 = _grouped_ffn(be1, buf1, W1, b1, W2, b2)
    g0 = _sc_gather(y0, p0h)
    g1 = _sc_gather(y1, p1h)
    out0 = _combine(g0, w0, w1, 0)
    out1 = _combine(g1, w0, w1, 1)
    out = jnp.concatenate([out0, out1])
    return out.reshape(B, S, D)


# speedup vs baseline: 1.1528x; 1.1528x over previous
import functools

import jax
import jax.numpy as jnp
from jax import lax
from jax.experimental import pallas as pl
from jax.experimental.pallas import tpu as pltpu
from jax.experimental.pallas import tpu_sc as plsc


def _pack_bf16(a):
    u = lax.bitcast_convert_type(a.astype(jnp.bfloat16), jnp.uint16)
    m = u.shape[1] // 2
    lo = u[:, :m].astype(jnp.int32)
    hi = u[:, m:].astype(jnp.int32)
    return lo | (hi << 16)


def _unpack_bf16(p):
    lo = (p & 0xFFFF).astype(jnp.uint16)
    hi = lax.shift_right_logical(p, 16).astype(jnp.uint16)
    return jnp.concatenate([lax.bitcast_convert_type(lo, jnp.bfloat16),
                            lax.bitcast_convert_type(hi, jnp.bfloat16)],
                           axis=1)


D_MODEL = 1024
D_EXPERT = 512
NUM_EXPERTS = 8
TOPK = 2
T = 8192
NH = 1
TH = T // NH
PH = TH * TOPK
BT = 512
BC = 512
NBLK = PH // BC + NUM_EXPERTS
BUF = NBLK * BC
NW = 32
PW = PH // NW
CH = 64
HB = TH // BT


def _router_block(x_ref, wg_ref, e0_ref, e1_ref, r0_ref, r1_ref,
                  w0_ref, w1_ref, cnt_ref, xb16_ref, crun_ref):
    i = pl.program_id(0)

    @pl.when(lax.rem(i, HB) == 0)
    def _():
        crun_ref[...] = jnp.zeros((8, NUM_EXPERTS), jnp.float32)

    xb = x_ref[...]
    logits = lax.dot_general(xb, wg_ref[...], (((1,), (1,)), ((), ())),
                             preferred_element_type=jnp.float32)
    m = jnp.max(logits, axis=-1, keepdims=True)
    ex = jnp.exp(logits - m)
    probs = ex / jnp.sum(ex, axis=-1, keepdims=True)

    e0 = jnp.argmax(probs, axis=-1)
    w0 = jnp.max(probs, axis=-1)
    iota = lax.broadcasted_iota(jnp.int32, probs.shape, 1)
    probs2 = jnp.where(iota == e0[:, None], -jnp.inf, probs)
    e1 = jnp.argmax(probs2, axis=-1)
    w1 = jnp.max(probs2, axis=-1)
    s = w0 + w1

    oh0 = (iota == e0[:, None]).astype(jnp.float32)
    oh1 = (iota == e1[:, None]).astype(jnp.float32)
    ri = lax.broadcasted_iota(jnp.int32, (BT, BT), 0)
    ci = lax.broadcasted_iota(jnp.int32, (BT, BT), 1)
    tri = (ri > ci).astype(jnp.float32)
    ex0 = lax.dot_general(tri, oh0, (((1,), (0,)), ((), ())),
                          preferred_element_type=jnp.float32)
    ex1 = lax.dot_general(tri, oh1, (((1,), (0,)), ((), ())),
                          preferred_element_type=jnp.float32)
    cnt0 = jnp.sum(oh0, axis=0)
    cnt1 = jnp.sum(oh1, axis=0)
    crun = crun_ref[0:1, :]
    r0 = jnp.sum(oh0 * (crun + ex0), axis=1)
    r1 = jnp.sum(oh1 * (crun + cnt0[None, :] + ex1), axis=1)
    new = crun[0] + cnt0 + cnt1
    crun_ref[...] = jnp.broadcast_to(new[None, :], (8, NUM_EXPERTS))
    cnt_ref[...] = jnp.broadcast_to(new[None, :], (8, NUM_EXPERTS))

    e0_ref[...] = e0[:, None]
    e1_ref[...] = e1[:, None]
    r0_ref[...] = r0.astype(jnp.int32)[:, None]
    r1_ref[...] = r1.astype(jnp.int32)[:, None]
    w0_ref[...] = (w0 / s)[:, None]
    w1_ref[...] = (w1 / s)[:, None]
    xb16_ref[...] = _pack_bf16(xb)


def _router(xf, Wg):
    shapes = [
        jax.ShapeDtypeStruct((T, 1), jnp.int32),
        jax.ShapeDtypeStruct((T, 1), jnp.int32),
        jax.ShapeDtypeStruct((T, 1), jnp.int32),
        jax.ShapeDtypeStruct((T, 1), jnp.int32),
        jax.ShapeDtypeStruct((T, 1), jnp.float32),
        jax.ShapeDtypeStruct((T, 1), jnp.float32),
        jax.ShapeDtypeStruct((8 * NH, NUM_EXPERTS), jnp.float32),
        jax.ShapeDtypeStruct((T, D_MODEL // 2), jnp.int32),
    ]
    tspec = pl.BlockSpec((BT, 1), lambda i: (i, 0))
    return pl.pallas_call(
        _router_block,
        grid=(T // BT,),
        in_specs=[
            pl.BlockSpec((BT, D_MODEL), lambda i: (i, 0)),
            pl.BlockSpec((NUM_EXPERTS, D_MODEL), lambda i: (0, 0)),
        ],
        out_specs=[tspec, tspec, tspec, tspec, tspec, tspec,
                   pl.BlockSpec((8, NUM_EXPERTS), lambda i: (i // HB, 0)),
                   pl.BlockSpec((BT, D_MODEL // 2), lambda i: (i, 0))],
        out_shape=shapes,
        scratch_shapes=[pltpu.VMEM((8, NUM_EXPERTS), jnp.float32)],
    )(xf, Wg)


def _sc_scatter(xp, p, h):
    mesh = plsc.VectorSubcoreMesh(core_axis_name="c", subcore_axis_name="s")

    nchunks = PW // CH

    @functools.partial(
        pl.kernel, mesh=mesh,
        out_type=jax.ShapeDtypeStruct((BUF, D_MODEL // 2), jnp.int32),
        scratch_types=[pltpu.VMEM((CH,), jnp.int32),
                       pltpu.VMEM((CH,), jnp.int32),
                       pltpu.VMEM((CH, D_MODEL // 2), jnp.int32),
                       pltpu.VMEM((CH, D_MODEL // 2), jnp.int32),
                       pltpu.SemaphoreType.DMA,
                       pltpu.SemaphoreType.DMA,
                       pltpu.SemaphoreType.DMA,
                       pltpu.SemaphoreType.DMA,
                       pltpu.SemaphoreType.DMA,
                       pltpu.SemaphoreType.DMA],
    )
    def k(x_hbm, p_hbm, buf_hbm, i0, i1, d0, d1,
          si0, si1, sd0, sd1, so0, so1):
        wid = lax.axis_index("s") * 2 + lax.axis_index("c")
        base = wid * PW
        trow = h * TH + lax.rem(base, TH)
        idx = (i0, i1)
        dat = (d0, d1)
        sis = (si0, si1)
        sds = (sd0, sd1)
        sos = (so0, so1)

        def load(c):
            b = c & 1
            hi = pltpu.async_copy(p_hbm.at[pl.ds(base + c * CH, CH)],
                                  idx[b], sis[b])
            hd = pltpu.async_copy(x_hbm.at[pl.ds(trow + c * CH, CH)],
                                  dat[b], sds[b])
            return hi, hd

        hin = [None] * nchunks
        hout = [None] * nchunks
        hin[0] = load(0)
        for c in range(nchunks):
            b = c & 1
            hin[c][0].wait()
            hin[c][1].wait()
            hout[c] = pltpu.async_copy(dat[b], buf_hbm.at[idx[b]], sos[b])
            if c + 1 < nchunks:
                if c >= 1:
                    hout[c - 1].wait()
                hin[c + 1] = load(c + 1)
        hout[nchunks - 1].wait()
        if nchunks >= 2:
            hout[nchunks - 2].wait()

    return k(xp, p)


def _sc_gather(y, p):
    mesh = plsc.VectorSubcoreMesh(core_axis_name="c", subcore_axis_name="s")

    nchunks = PW // CH

    @functools.partial(
        pl.kernel, mesh=mesh,
        out_type=jax.ShapeDtypeStruct((PH, D_MODEL // 2), jnp.int32),
        scratch_types=[pltpu.VMEM((CH,), jnp.int32),
                       pltpu.VMEM((CH,), jnp.int32),
                       pltpu.VMEM((CH, D_MODEL // 2), jnp.int32),
                       pltpu.VMEM((CH, D_MODEL // 2), jnp.int32),
                       pltpu.SemaphoreType.DMA,
                       pltpu.SemaphoreType.DMA,
                       pltpu.SemaphoreType.DMA,
                       pltpu.SemaphoreType.DMA,
                       pltpu.SemaphoreType.DMA,
                       pltpu.SemaphoreType.DMA],
    )
    def k(y_hbm, p_hbm, g_hbm, i0, i1, r0, r1,
          si0, si1, sg0, sg1, ss0, ss1):
        wid = lax.axis_index("s") * 2 + lax.axis_index("c")
        base = wid * PW
        idx = (i0, i1)
        rows = (r0, r1)
        sis = (si0, si1)
        sgs = (sg0, sg1)
        sss = (ss0, ss1)

        def load_idx(c):
            b = c & 1
            return pltpu.async_copy(p_hbm.at[pl.ds(base + c * CH, CH)],
                                    idx[b], sis[b])

        hidx = [None] * nchunks
        hg = [None] * nchunks
        hs = [None] * nchunks
        hidx[0] = load_idx(0)
        for c in range(nchunks):
            b = c & 1
            hidx[c].wait()
            if c >= 2:
                hs[c - 2].wait()
            hg[c] = pltpu.async_copy(y_hbm.at[idx[b]], rows[b], sgs[b])
            if c + 1 < nchunks:
                hidx[c + 1] = load_idx(c + 1)
            hg[c].wait()
            hs[c] = pltpu.async_copy(rows[b],
                                     g_hbm.at[pl.ds(base + c * CH, CH)],
                                     sss[b])
        hs[nchunks - 1].wait()
        if nchunks >= 2:
            hs[nchunks - 2].wait()

    return k(y, p)


def _ffn_block(be_ref, buf_ref, w1_ref, b1_ref, w2_ref, b2_ref, y_ref):
    xb = _unpack_bf16(buf_ref[...])
    h = lax.dot_general(xb, w1_ref[0].astype(jnp.bfloat16),
                        (((1,), (1,)), ((), ())),
                        preferred_element_type=jnp.float32) + b1_ref[0]
    h = jnp.maximum(h, 0.0).astype(jnp.bfloat16)
    y = lax.dot_general(h, w2_ref[0].astype(jnp.bfloat16),
                        (((1,), (1,)), ((), ())),
                        preferred_element_type=jnp.float32) + b2_ref[0]
    y_ref[...] = _pack_bf16(y)


def _grouped_ffn(be, buf, W1, b1, W2, b2):
    grid_spec = pltpu.PrefetchScalarGridSpec(
        num_scalar_prefetch=1,
        grid=(NBLK,),
        in_specs=[
            pl.BlockSpec((BC, D_MODEL // 2), lambda i, be: (i, 0)),
            pl.BlockSpec((1, D_EXPERT, D_MODEL), lambda i, be: (be[i], 0, 0)),
            pl.BlockSpec((1, 1, D_EXPERT), lambda i, be: (be[i], 0, 0)),
            pl.BlockSpec((1, D_MODEL, D_EXPERT), lambda i, be: (be[i], 0, 0)),
            pl.BlockSpec((1, 1, D_MODEL), lambda i, be: (be[i], 0, 0)),
        ],
        out_specs=pl.BlockSpec((BC, D_MODEL // 2), lambda i, be: (i, 0)),
    )
    return pl.pallas_call(
        _ffn_block,
        grid_spec=grid_spec,
        out_shape=jax.ShapeDtypeStruct((BUF, D_MODEL // 2), jnp.int32),
    )(be, buf, W1, b1.reshape(NUM_EXPERTS, 1, D_EXPERT),
      W2, b2.reshape(NUM_EXPERTS, 1, D_MODEL))


def _combine_block(ya_ref, yb_ref, w0_ref, w1_ref, out_ref):
    ya = _unpack_bf16(ya_ref[...]).astype(jnp.float32)
    yb = _unpack_bf16(yb_ref[...]).astype(jnp.float32)
    out_ref[...] = ya * w0_ref[...] + yb * w1_ref[...]


def _combine(g, w0, w1, h):
    nb = TH // BT
    return pl.pallas_call(
        _combine_block,
        grid=(nb,),
        in_specs=[
            pl.BlockSpec((BT, D_MODEL // 2), lambda i: (i, 0)),
            pl.BlockSpec((BT, D_MODEL // 2), lambda i: (i + nb, 0)),
            pl.BlockSpec((BT, 1), lambda i: (i + h * nb, 0)),
            pl.BlockSpec((BT, 1), lambda i: (i + h * nb, 0)),
        ],
        out_specs=pl.BlockSpec((BT, D_MODEL), lambda i: (i, 0)),
        out_shape=jax.ShapeDtypeStruct((TH, D_MODEL), jnp.float32),
    )(g, g, w0, w1)


def _half_plumbing(counts, e0, e1, r0, r1, h):
    cnt = counts[8 * h].astype(jnp.int32)
    padded = ((cnt + BC - 1) // BC) * BC
    ends = jnp.cumsum(padded)
    off = ends - padded
    sl = slice(h * TH, (h + 1) * TH)
    p0 = jnp.take(off, e0[sl, 0]) + r0[sl, 0]
    p1 = jnp.take(off, e1[sl, 0]) + r1[sl, 0]
    p = jnp.concatenate([p0, p1])
    starts = jnp.arange(NBLK, dtype=jnp.int32) * BC
    be = jnp.minimum(
        jnp.searchsorted(ends, starts, side="right").astype(jnp.int32),
        NUM_EXPERTS - 1)
    return p, be


@jax.jit
def kernel(x, Wg, W1, b1, W2, b2):
    B, S, D = x.shape
    xf = x.reshape(T, D)

    e0, e1, r0, r1, w0, w1, counts, xp = _router(xf, Wg)
    outs = []
    for h in range(NH):
        ph, beh = _half_plumbing(counts, e0, e1, r0, r1, h)
        buf = _sc_scatter(xp, ph, h)
        y = _grouped_ffn(beh, buf, W1, b1, W2, b2)
        g = _sc_gather(y, ph)
        outs.append(_combine(g, w0, w1, h))
    out = outs[0] if NH == 1 else jnp.concatenate(outs)
    return out.reshape(B, S, D)
